# baseline (speedup 1.0000x reference)
import functools

import numpy as np
import jax
import jax.numpy as jnp
from jax.experimental import pallas as pl
from jax.experimental.pallas import tpu as pltpu

MV = 16
RMS_EPS = 1e-6
ND_LANES = (0, 2, 3, 4, 8, 9, 10, 14)
TRI_LANES = (11, 12, 13)

_BLADES = [(), (0,), (1,), (2,), (3,), (0, 1), (0, 2), (0, 3), (1, 2), (1, 3),
           (2, 3), (0, 1, 2), (0, 1, 3), (0, 2, 3), (1, 2, 3), (0, 1, 2, 3)]
_B2I = {b: i for i, b in enumerate(_BLADES)}


def _perm_sign(seq):
    arr = list(seq)
    sgn = 1.0
    for a in range(1, len(arr)):
        b = a
        while b > 0 and arr[b - 1] > arr[b]:
            arr[b - 1], arr[b] = arr[b], arr[b - 1]
            sgn = -sgn
            b -= 1
    return sgn, arr


def _mul_blades(x, y):
    sgn, arr = _perm_sign(list(x) + list(y))
    out, i = [], 0
    while i < len(arr):
        if i + 1 < len(arr) and arr[i] == arr[i + 1]:
            if arr[i] == 0:
                return 0.0, ()
            i += 2
        else:
            out.append(arr[i])
            i += 1
    return sgn, tuple(out)


def _tables():
    gp = np.zeros((16, 16, 16), np.float32)
    wedge = np.zeros((16, 16, 16), np.float32)
    for i, a in enumerate(_BLADES):
        for j, b in enumerate(_BLADES):
            s, c = _mul_blades(a, b)
            if s:
                gp[i, j, _B2I[c]] = s
            if not (set(a) & set(b)):
                s2, arr = _perm_sign(list(a) + list(b))
                wedge[i, j, _B2I[tuple(arr)]] = s2
    dual = np.zeros((16, 16), np.float32)
    for i, bl in enumerate(_BLADES):
        comp = tuple(sorted(set((0, 1, 2, 3)) - set(bl)))
        s, _ = _perm_sign(list(bl) + list(comp))
        dual[_B2I[comp], i] = s
    join = np.einsum("mn,pqm,pi,qj->ijn", dual, wedge, dual, dual)
    return gp, join.astype(np.float32)


_GP_TBL, _JOIN_TBL = _tables()


def _term_list(tbl):
    out = [[] for _ in range(16)]
    for i, j, n in np.argwhere(tbl != 0.0):
        out[int(n)].append((int(i), int(j), float(tbl[i, j, n])))
    return out


_GP_TERMS = _term_list(_GP_TBL)
_JOIN_TERMS = _term_list(_JOIN_TBL)


def _block_kernel(x_ref, w_ref, b_ref, mask_ref, pq_ref, pk_ref, qc2_ref,
                  kc2_ref, wo_ref, bo_ref, wb_ref,
                  bb_ref, ps_ref, w1_ref, b1_ref, w2_ref, b2_ref, o_ref, *,
                  heads, cdim, inv_c, seq, scale, fq):
    bf = jnp.bfloat16
    x = x_ref[0]
    ms = jnp.sum(x * x * mask_ref[...], axis=-1, keepdims=True) * inv_c
    xn = (x * jax.lax.rsqrt(ms + RMS_EPS)).astype(bf)
    qkv = jnp.dot(xn, w_ref[...], preferred_element_type=jnp.float32)
    qkv = qkv + b_ref[...]

    row = jax.lax.broadcasted_iota(jnp.int32, (seq, seq), 0)
    col = jax.lax.broadcasted_iota(jnp.int32, (seq, seq), 1)
    causal = col <= row

    acc = None
    for h in range(heads):
        q = qkv[:, h * fq:(h + 1) * fq]
        k = qkv[:, (heads + h) * fq:(heads + h + 1) * fq]
        v = qkv[:, 2 * heads * fq + h * cdim:2 * heads * fq + (h + 1) * cdim]
        injq = jnp.dot((q * q).astype(bf), pq_ref[h],
                       preferred_element_type=jnp.float32) + qc2_ref[h][None]
        injk = jnp.dot((k * k).astype(bf), pk_ref[...],
                       preferred_element_type=jnp.float32) + kc2_ref[...]
        qf = jnp.concatenate([q.astype(bf), injq.astype(bf)], axis=1)
        kf = jnp.concatenate([k.astype(bf), injk.astype(bf)], axis=1)
        s = jax.lax.dot_general(qf, kf, (((1,), (1,)), ((), ())),
                                preferred_element_type=jnp.float32) * scale
        s = jnp.where(causal, s, -1e30)
        m = jnp.max(s, axis=-1, keepdims=True)
        p = jnp.exp(s - m)
        l = jnp.sum(p, axis=-1, keepdims=True)
        attn_h = jnp.dot(p.astype(bf), v.astype(bf),
                         preferred_element_type=jnp.float32) / l
        part = jnp.dot(attn_h.astype(bf), wo_ref[h],
                       preferred_element_type=jnp.float32)
        acc = part if acc is None else acc + part

    xa = acc + bo_ref[...] + x
    ms2 = jnp.sum(xa * xa * mask_ref[...], axis=-1, keepdims=True) * inv_c
    xn2 = (xa * jax.lax.rsqrt(ms2 + RMS_EPS)).astype(bf)
    y = jnp.dot(xn2, wb_ref[...], preferred_element_type=jnp.float32)
    y = y + bb_ref[...]

    odim = wb_ref.shape[1] // 4
    c_i = odim // MV

    def to_bm(kk):
        tt = jnp.transpose(y[:, kk * odim:(kk + 1) * odim].astype(bf))
        return tt.reshape(c_i, MV, seq).astype(jnp.float32)

    lg, rg, rj = to_bm(0), to_bm(1), to_bm(3)
    lj = to_bm(2) * ps_ref[0, 0, 0]

    halves = []
    for terms, a, bb2 in ((_GP_TERMS, lg, rg), (_JOIN_TERMS, lj, rj)):
        outs = []
        for n in range(16):
            nacc = None
            for (i, j, sgn) in terms[n]:
                t = a[:, i, :] * bb2[:, j, :]
                if sgn == -1.0:
                    t = -t
                elif sgn != 1.0:
                    t = t * sgn
                nacc = t if nacc is None else nacc + t
            outs.append(nacc if nacc is not None
                        else jnp.zeros_like(a[:, 0, :]))
        half = jnp.stack(outs, axis=1)
        half = jnp.transpose(half.reshape(odim, seq).astype(bf))
        halves.append(half)
    z = jnp.concatenate(halves, axis=1)

    t2 = jnp.dot(z, w1_ref[...], preferred_element_type=jnp.float32)
    t2 = t2 + b1_ref[...]
    z2 = t2[:, :cdim]
    gate = jax.nn.gelu(t2[:, cdim:], approximate=True)
    gated = (z2 * gate).astype(bf)
    out = jnp.dot(gated, w2_ref[...], preferred_element_type=jnp.float32)
    o_ref[0] = out + b2_ref[...] + xa


def _feature_repack(qkv_w, qkv_b, w_ipa, w_daa, c_h):
    heads = w_ipa.shape[0]
    cdim = c_h * MV
    fq = 12 * c_h
    blades = list(ND_LANES) + list(TRI_LANES) + [0]
    idx = np.array([c * MV + bl for c in range(c_h) for bl in blades],
                   np.int32)
    nd12 = np.array([1.0] * 8 + [0.0] * 4, np.float32)
    tri12 = np.array([0.0] * 8 + [1.0] * 3 + [0.0], np.float32)
    qsc = (w_ipa[:, :, None] * nd12 + 2.0 * w_daa[:, :, None] * tri12)
    qsc = qsc.reshape(heads, fq)
    ksc = jnp.asarray(np.tile(nd12 + tri12, c_h))

    qcols, kcols, qbias, kbias = [], [], [], []
    for h in range(heads):
        qcols.append(qkv_w[:, h * cdim + idx] * qsc[h][None, :])
        kcols.append(qkv_w[:, (heads + h) * cdim + idx] * ksc[None, :])
        qbias.append(qkv_b[:, h * cdim + idx] * qsc[h][None, :])
        kbias.append(qkv_b[:, (heads + h) * cdim + idx] * ksc[None, :])
    vcols = qkv_w[:, 2 * heads * cdim:]
    vbias = qkv_b[:, 2 * heads * cdim:]
    w_new = jnp.concatenate(qcols + kcols + [vcols], axis=1)
    b_new = jnp.concatenate(qbias + kbias + [vbias], axis=1)

    patq = np.zeros((fq, 2 * c_h), np.float32)
    patk = np.zeros((fq, 2 * c_h), np.float32)
    for c in range(c_h):
        for j in (8, 9, 10):
            patq[c * 12 + j, 2 * c] = 1.0
            patk[c * 12 + j, 2 * c + 1] = 1.0
    colw = jnp.repeat(-0.25 / w_daa, 2, axis=1)
    laneA = np.tile(np.array([1.0, 0.0], np.float32), c_h)
    laneB = np.tile(np.array([0.0, 1.0], np.float32), c_h)
    pq = (jnp.asarray(patq)[None] * colw[:, None, :]).astype(jnp.bfloat16)
    pk = jnp.asarray(patk).astype(jnp.bfloat16)
    qc2 = (-w_daa[:, :, None] * jnp.ones((1, 1, 2))).reshape(heads, 2 * c_h)
    qc2 = qc2 * jnp.asarray(laneB)[None, :]
    kc2 = jnp.asarray(laneA).reshape(1, 2 * c_h)
    return w_new, b_new, pq, pk, qc2, kc2, fq


def kernel(x, ref_input, qkv_w, qkv_b, attn_out_w, attn_out_b, bil_w, bil_b,
           bil_out_w, bil_out_b, mlp_out_w, mlp_out_b, w_ipa, w_daa,
           norm_mask):
    b, t, c_h, mv = x.shape
    assert mv == MV
    heads = w_ipa.shape[0]
    cdim = c_h * MV
    c_inter = bil_w.shape[1] // (4 * MV)
    inv_c = 1.0 / c_h
    scale = 1.0 / np.sqrt(c_h * 13)

    x3 = x.reshape(b, t, cdim)
    wq_new, bq_new, pq, pk, qc2, kc2, fq = _feature_repack(
        qkv_w, qkv_b, w_ipa, w_daa, c_h)
    ref_ps = jnp.broadcast_to(ref_input[:, 0, 0, 15][:, None, None],
                              (b, 1, 128)).astype(jnp.float32)
    w1 = jnp.concatenate(
        [bil_out_w, jnp.repeat(bil_out_w[:, ::MV], MV, axis=1)], axis=1)
    b1 = jnp.concatenate(
        [bil_out_b, jnp.repeat(bil_out_b[:, ::MV], MV, axis=1)], axis=1)
    zdim = 2 * c_inter * MV

    bf = jnp.bfloat16
    out = pl.pallas_call(
        functools.partial(_block_kernel, heads=heads, cdim=cdim, inv_c=inv_c,
                          seq=t, scale=scale, fq=fq),
        out_shape=jax.ShapeDtypeStruct((b, t, cdim), jnp.float32),
        grid=(b,),
        in_specs=[
            pl.BlockSpec((1, t, cdim), lambda i: (i, 0, 0)),
            pl.BlockSpec(wq_new.shape, lambda i: (0, 0)),
            pl.BlockSpec(bq_new.shape, lambda i: (0, 0)),
            pl.BlockSpec(norm_mask.shape, lambda i: (0, 0)),
            pl.BlockSpec(pq.shape, lambda i: (0, 0, 0)),
            pl.BlockSpec(pk.shape, lambda i: (0, 0)),
            pl.BlockSpec(qc2.shape, lambda i: (0, 0)),
            pl.BlockSpec(kc2.shape, lambda i: (0, 0)),
            pl.BlockSpec((heads, cdim, cdim), lambda i: (0, 0, 0)),
            pl.BlockSpec(attn_out_b.shape, lambda i: (0, 0)),
            pl.BlockSpec(bil_w.shape, lambda i: (0, 0)),
            pl.BlockSpec(bil_b.shape, lambda i: (0, 0)),
            pl.BlockSpec((1, 1, 128), lambda i: (i, 0, 0)),
            pl.BlockSpec((zdim, 2 * cdim), lambda i: (0, 0)),
            pl.BlockSpec((1, 2 * cdim), lambda i: (0, 0)),
            pl.BlockSpec((cdim, cdim), lambda i: (0, 0)),
            pl.BlockSpec(mlp_out_b.shape, lambda i: (0, 0)),
        ],
        out_specs=pl.BlockSpec((1, t, cdim), lambda i: (i, 0, 0)),
        compiler_params=pltpu.CompilerParams(
            dimension_semantics=("parallel",),
            vmem_limit_bytes=100 * 1024 * 1024),
    )(x3, wq_new.astype(bf), bq_new, norm_mask, pq, pk, qc2, kc2,
      attn_out_w.reshape(heads, cdim, cdim).astype(bf), attn_out_b,
      bil_w.astype(bf), bil_b, ref_ps, w1.astype(bf), b1,
      mlp_out_w.astype(bf), mlp_out_b)

    return out.reshape(b, t, c_h, MV)


# speedup vs baseline: 1.1421x; 1.1421x over previous
import functools

import numpy as np
import jax
import jax.numpy as jnp
from jax.experimental import pallas as pl
from jax.experimental.pallas import tpu as pltpu

MV = 16
RMS_EPS = 1e-6
ND_LANES = (0, 2, 3, 4, 8, 9, 10, 14)
TRI_LANES = (11, 12, 13)

_BLADES = [(), (0,), (1,), (2,), (3,), (0, 1), (0, 2), (0, 3), (1, 2), (1, 3),
           (2, 3), (0, 1, 2), (0, 1, 3), (0, 2, 3), (1, 2, 3), (0, 1, 2, 3)]
_B2I = {b: i for i, b in enumerate(_BLADES)}


def _perm_sign(seq):
    arr = list(seq)
    sgn = 1.0
    for a in range(1, len(arr)):
        b = a
        while b > 0 and arr[b - 1] > arr[b]:
            arr[b - 1], arr[b] = arr[b], arr[b - 1]
            sgn = -sgn
            b -= 1
    return sgn, arr


def _mul_blades(x, y):
    sgn, arr = _perm_sign(list(x) + list(y))
    out, i = [], 0
    while i < len(arr):
        if i + 1 < len(arr) and arr[i] == arr[i + 1]:
            if arr[i] == 0:
                return 0.0, ()
            i += 2
        else:
            out.append(arr[i])
            i += 1
    return sgn, tuple(out)


def _tables():
    gp = np.zeros((16, 16, 16), np.float32)
    wedge = np.zeros((16, 16, 16), np.float32)
    for i, a in enumerate(_BLADES):
        for j, b in enumerate(_BLADES):
            s, c = _mul_blades(a, b)
            if s:
                gp[i, j, _B2I[c]] = s
            if not (set(a) & set(b)):
                s2, arr = _perm_sign(list(a) + list(b))
                wedge[i, j, _B2I[tuple(arr)]] = s2
    dual = np.zeros((16, 16), np.float32)
    for i, bl in enumerate(_BLADES):
        comp = tuple(sorted(set((0, 1, 2, 3)) - set(bl)))
        s, _ = _perm_sign(list(bl) + list(comp))
        dual[_B2I[comp], i] = s
    join = np.einsum("mn,pqm,pi,qj->ijn", dual, wedge, dual, dual)
    return gp, join.astype(np.float32)


_GP_TBL, _JOIN_TBL = _tables()


def _term_list(tbl):
    out = [[] for _ in range(16)]
    for i, j, n in np.argwhere(tbl != 0.0):
        out[int(n)].append((int(i), int(j), float(tbl[i, j, n])))
    return out


_GP_TERMS = _term_list(_GP_TBL)
_JOIN_TERMS = _term_list(_JOIN_TBL)


def _block_kernel(x_ref, w_ref, b_ref, mask_ref, qs_ref, qc_ref, ks_ref,
                  kc_ref, inj_ref, m1_ref, m5_ref, wo_ref, bo_ref, wb_ref,
                  bb_ref, ps_ref, w1_ref, b1_ref, w2_ref, b2_ref, o_ref, *,
                  heads, cdim, inv_c, seq, scale):
    bf = jnp.bfloat16
    x = x_ref[0]
    ms = jnp.sum(x * x * mask_ref[...], axis=-1, keepdims=True) * inv_c
    xn = (x * jax.lax.rsqrt(ms + RMS_EPS)).astype(bf)
    qkv = jnp.dot(xn, w_ref[...], preferred_element_type=jnp.float32)
    qkv = qkv + b_ref[...]

    half = seq // 2
    row = jax.lax.broadcasted_iota(jnp.int32, (half, half), 0)
    col = jax.lax.broadcasted_iota(jnp.int32, (half, half), 1)
    causal_ll = col <= row

    def masked_softmax(s, causal):
        s = jnp.where(causal, s, -1e30)
        m = jnp.max(s, axis=-1, keepdims=True)
        p = jnp.exp(s - m)
        l = jnp.sum(p, axis=-1, keepdims=True)
        return p.astype(bf), l

    acc = None
    for h in range(heads):
        q = qkv[:, h * cdim:(h + 1) * cdim]
        k = qkv[:, (heads + h) * cdim:(heads + h + 1) * cdim]
        v = qkv[:, (2 * heads + h) * cdim:(2 * heads + h + 1) * cdim]
        sq = jnp.concatenate([q * q, k * k], axis=1).astype(bf)
        inj = jnp.dot(sq, inj_ref[h], preferred_element_type=jnp.float32)
        qf = (q * qs_ref[h][None, :] + qc_ref[h][None, :]
              + inj * m1_ref[...]).astype(bf)
        kf = (k * ks_ref[...] + kc_ref[...] + inj * m5_ref[...]).astype(bf)
        vb = v.astype(bf)
        qf_lo, qf_hi = qf[:half], qf[half:]
        kf_lo = kf[:half]
        s_lo = jax.lax.dot_general(qf_lo, kf_lo, (((1,), (1,)), ((), ())),
                                   preferred_element_type=jnp.float32) * scale
        p_lo, l_lo = masked_softmax(s_lo, causal_ll)
        a_lo = jnp.dot(p_lo, vb[:half],
                       preferred_element_type=jnp.float32) / l_lo
        s_hi = jax.lax.dot_general(qf_hi, kf, (((1,), (1,)), ((), ())),
                                   preferred_element_type=jnp.float32) * scale
        hi_mask = jnp.concatenate(
            [jnp.ones((half, half), jnp.bool_), causal_ll], axis=1)
        p_hi, l_hi = masked_softmax(s_hi, hi_mask)
        a_hi = jnp.dot(p_hi, vb, preferred_element_type=jnp.float32) / l_hi
        attn_h = jnp.concatenate([a_lo, a_hi], axis=0)
        part = jnp.dot(attn_h.astype(bf), wo_ref[h],
                       preferred_element_type=jnp.float32)
        acc = part if acc is None else acc + part

    xa = acc + bo_ref[...] + x
    ms2 = jnp.sum(xa * xa * mask_ref[...], axis=-1, keepdims=True) * inv_c
    xn2 = (xa * jax.lax.rsqrt(ms2 + RMS_EPS)).astype(bf)
    y = jnp.dot(xn2, wb_ref[...], preferred_element_type=jnp.float32)
    y = y + bb_ref[...]

    odim = wb_ref.shape[1] // 4
    c_i = odim // MV

    def to_bm(kk):
        tt = jnp.transpose(y[:, kk * odim:(kk + 1) * odim].astype(bf))
        return tt.reshape(c_i, MV, seq).astype(jnp.float32)

    lg, rg, rj = to_bm(0), to_bm(1), to_bm(3)
    lj = to_bm(2) * ps_ref[0, 0, 0]

    halves = []
    for terms, a, bb2 in ((_GP_TERMS, lg, rg), (_JOIN_TERMS, lj, rj)):
        outs = []
        for n in range(16):
            nacc = None
            for (i, j, sgn) in terms[n]:
                t = a[:, i, :] * bb2[:, j, :]
                if sgn == -1.0:
                    t = -t
                elif sgn != 1.0:
                    t = t * sgn
                nacc = t if nacc is None else nacc + t
            outs.append(nacc if nacc is not None
                        else jnp.zeros_like(a[:, 0, :]))
        half = jnp.stack(outs, axis=1)
        half = jnp.transpose(half.reshape(odim, seq).astype(bf))
        halves.append(half)
    z = jnp.concatenate(halves, axis=1)

    t2 = jnp.dot(z, w1_ref[...], preferred_element_type=jnp.float32)
    t2 = t2 + b1_ref[...]
    z2 = t2[:, :cdim]
    gate = jax.nn.gelu(t2[:, cdim:], approximate=True)
    gated = (z2 * gate).astype(bf)
    out = jnp.dot(gated, w2_ref[...], preferred_element_type=jnp.float32)
    o_ref[0] = out + b2_ref[...] + xa


def _feature_constants(w_ipa, w_daa, c_h):
    heads = w_ipa.shape[0]
    cdim = c_h * MV
    nd = np.zeros((MV,), np.float32)
    nd[list(ND_LANES)] = 1.0
    tri = np.zeros((MV,), np.float32)
    tri[list(TRI_LANES)] = 1.0
    lane1 = np.zeros((MV,), np.float32)
    lane1[1] = 1.0
    lane5 = np.zeros((MV,), np.float32)
    lane5[5] = 1.0

    ndj = jnp.asarray(nd)
    trij = jnp.asarray(tri)
    qscale = (w_ipa[:, :, None] * ndj + 2.0 * w_daa[:, :, None] * trij)
    qscale = qscale.reshape(heads, cdim)
    qconst = (-w_daa[:, :, None] * jnp.asarray(lane5)).reshape(heads, cdim)
    kscale = np.tile(nd + tri, c_h).reshape(1, cdim)
    kconst = np.tile(lane1, c_h).reshape(1, cdim)

    pat1 = np.zeros((cdim, cdim), np.float32)
    pat5 = np.zeros((cdim, cdim), np.float32)
    for c in range(c_h):
        for t in TRI_LANES:
            pat1[c * MV + t, c * MV + 1] = 1.0
            pat5[c * MV + t, c * MV + 5] = 1.0
    col_w = (-w_daa[:, :, None] * jnp.ones((1, 1, MV))).reshape(heads, 1, cdim)
    top = jnp.asarray(pat1)[None] * col_w
    bot = jnp.broadcast_to(jnp.asarray(pat5)[None], top.shape)
    inj = jnp.concatenate([top, bot], axis=1).astype(jnp.bfloat16)
    m1 = np.tile(lane1, c_h).reshape(1, cdim)
    m5 = np.tile(lane5, c_h).reshape(1, cdim)
    return (qscale, qconst, jnp.asarray(kscale), jnp.asarray(kconst),
            inj, jnp.asarray(m1), jnp.asarray(m5))


def kernel(x, ref_input, qkv_w, qkv_b, attn_out_w, attn_out_b, bil_w, bil_b,
           bil_out_w, bil_out_b, mlp_out_w, mlp_out_b, w_ipa, w_daa,
           norm_mask):
    b, t, c_h, mv = x.shape
    assert mv == MV
    heads = w_ipa.shape[0]
    cdim = c_h * MV
    c_inter = bil_w.shape[1] // (4 * MV)
    inv_c = 1.0 / c_h
    scale = 1.0 / np.sqrt(c_h * 13)

    x3 = x.reshape(b, t, cdim)
    qs, qc, ks, kc, inj, m1, m5 = _feature_constants(w_ipa, w_daa, c_h)
    ref_ps = jnp.broadcast_to(ref_input[:, 0, 0, 15][:, None, None],
                              (b, 1, 128)).astype(jnp.float32)
    w1 = jnp.concatenate(
        [bil_out_w, jnp.repeat(bil_out_w[:, ::MV], MV, axis=1)], axis=1)
    b1 = jnp.concatenate(
        [bil_out_b, jnp.repeat(bil_out_b[:, ::MV], MV, axis=1)], axis=1)
    zdim = 2 * c_inter * MV

    bf = jnp.bfloat16
    out = pl.pallas_call(
        functools.partial(_block_kernel, heads=heads, cdim=cdim, inv_c=inv_c,
                          seq=t, scale=scale),
        out_shape=jax.ShapeDtypeStruct((b, t, cdim), jnp.float32),
        grid=(b,),
        in_specs=[
            pl.BlockSpec((1, t, cdim), lambda i: (i, 0, 0)),
            pl.BlockSpec(qkv_w.shape, lambda i: (0, 0)),
            pl.BlockSpec(qkv_b.shape, lambda i: (0, 0)),
            pl.BlockSpec(norm_mask.shape, lambda i: (0, 0)),
            pl.BlockSpec((heads, cdim), lambda i: (0, 0)),
            pl.BlockSpec((heads, cdim), lambda i: (0, 0)),
            pl.BlockSpec((1, cdim), lambda i: (0, 0)),
            pl.BlockSpec((1, cdim), lambda i: (0, 0)),
            pl.BlockSpec((heads, 2 * cdim, cdim), lambda i: (0, 0, 0)),
            pl.BlockSpec((1, cdim), lambda i: (0, 0)),
            pl.BlockSpec((1, cdim), lambda i: (0, 0)),
            pl.BlockSpec((heads, cdim, cdim), lambda i: (0, 0, 0)),
            pl.BlockSpec(attn_out_b.shape, lambda i: (0, 0)),
            pl.BlockSpec(bil_w.shape, lambda i: (0, 0)),
            pl.BlockSpec(bil_b.shape, lambda i: (0, 0)),
            pl.BlockSpec((1, 1, 128), lambda i: (i, 0, 0)),
            pl.BlockSpec((zdim, 2 * cdim), lambda i: (0, 0)),
            pl.BlockSpec((1, 2 * cdim), lambda i: (0, 0)),
            pl.BlockSpec((cdim, cdim), lambda i: (0, 0)),
            pl.BlockSpec(mlp_out_b.shape, lambda i: (0, 0)),
        ],
        out_specs=pl.BlockSpec((1, t, cdim), lambda i: (i, 0, 0)),
        compiler_params=pltpu.CompilerParams(
            dimension_semantics=("parallel",),
            vmem_limit_bytes=100 * 1024 * 1024),
    )(x3, qkv_w.astype(bf), qkv_b, norm_mask, qs, qc, ks, kc, inj, m1, m5,
      attn_out_w.reshape(heads, cdim, cdim).astype(bf), attn_out_b,
      bil_w.astype(bf), bil_b, ref_ps, w1.astype(bf), b1,
      mlp_out_w.astype(bf), mlp_out_b)

    return out.reshape(b, t, c_h, MV)


# speedup vs baseline: 1.5114x; 1.3234x over previous
import functools

import numpy as np
import jax
import jax.numpy as jnp
from jax.experimental import pallas as pl
from jax.experimental.pallas import tpu as pltpu

MV = 16
RMS_EPS = 1e-6
ND_LANES = (0, 2, 3, 4, 8, 9, 10, 14)
TRI_LANES = (11, 12, 13)

_BLADES = [(), (0,), (1,), (2,), (3,), (0, 1), (0, 2), (0, 3), (1, 2), (1, 3),
           (2, 3), (0, 1, 2), (0, 1, 3), (0, 2, 3), (1, 2, 3), (0, 1, 2, 3)]
_B2I = {b: i for i, b in enumerate(_BLADES)}


def _perm_sign(seq):
    arr = list(seq)
    sgn = 1.0
    for a in range(1, len(arr)):
        b = a
        while b > 0 and arr[b - 1] > arr[b]:
            arr[b - 1], arr[b] = arr[b], arr[b - 1]
            sgn = -sgn
            b -= 1
    return sgn, arr


def _mul_blades(x, y):
    sgn, arr = _perm_sign(list(x) + list(y))
    out, i = [], 0
    while i < len(arr):
        if i + 1 < len(arr) and arr[i] == arr[i + 1]:
            if arr[i] == 0:
                return 0.0, ()
            i += 2
        else:
            out.append(arr[i])
            i += 1
    return sgn, tuple(out)


def _tables():
    gp = np.zeros((16, 16, 16), np.float32)
    wedge = np.zeros((16, 16, 16), np.float32)
    for i, a in enumerate(_BLADES):
        for j, b in enumerate(_BLADES):
            s, c = _mul_blades(a, b)
            if s:
                gp[i, j, _B2I[c]] = s
            if not (set(a) & set(b)):
                s2, arr = _perm_sign(list(a) + list(b))
                wedge[i, j, _B2I[tuple(arr)]] = s2
    dual = np.zeros((16, 16), np.float32)
    for i, bl in enumerate(_BLADES):
        comp = tuple(sorted(set((0, 1, 2, 3)) - set(bl)))
        s, _ = _perm_sign(list(bl) + list(comp))
        dual[_B2I[comp], i] = s
    join = np.einsum("mn,pqm,pi,qj->ijn", dual, wedge, dual, dual)
    return gp, join.astype(np.float32)


_GP_TBL, _JOIN_TBL = _tables()


def _term_list(tbl):
    out = [[] for _ in range(16)]
    for i, j, n in np.argwhere(tbl != 0.0):
        out[int(n)].append((int(i), int(j), float(tbl[i, j, n])))
    return out


_GP_TERMS = _term_list(_GP_TBL)
_JOIN_TERMS = _term_list(_JOIN_TBL)


def _block_kernel(x_ref, w_ref, b_ref, mask_ref, qs_ref, qc_ref, ks_ref,
                  kc_ref, inj_ref, m1_ref, m5_ref, wo_ref, bo_ref, wb_ref,
                  bb_ref, ps_ref, w1_ref, b1_ref, w2_ref, b2_ref, o_ref, *,
                  heads, cdim, inv_c, seq, scale):
    bf = jnp.bfloat16
    x = x_ref[0]
    ms = jnp.sum(x * x * mask_ref[...], axis=-1, keepdims=True) * inv_c
    xn = (x * jax.lax.rsqrt(ms + RMS_EPS)).astype(bf)
    qkv = jnp.dot(xn, w_ref[...], preferred_element_type=jnp.float32)
    qkv = qkv + b_ref[...]

    half = seq // 2
    row = jax.lax.broadcasted_iota(jnp.int32, (half, half), 0)
    col = jax.lax.broadcasted_iota(jnp.int32, (half, half), 1)
    causal_ll = col <= row

    def masked_softmax(s, causal):
        s = jnp.where(causal, s, -1e30)
        m = jnp.max(s, axis=-1, keepdims=True)
        p = jnp.exp(s - m)
        l = jnp.sum(p, axis=-1, keepdims=True)
        return p.astype(bf), l

    acc = None
    for h in range(heads):
        q = qkv[:, h * cdim:(h + 1) * cdim]
        k = qkv[:, (heads + h) * cdim:(heads + h + 1) * cdim]
        v = qkv[:, (2 * heads + h) * cdim:(2 * heads + h + 1) * cdim]
        sq = jnp.concatenate([q * q, k * k], axis=1).astype(bf)
        inj = jnp.dot(sq, inj_ref[h], preferred_element_type=jnp.float32)
        qf = (q * qs_ref[h][None, :] + qc_ref[h][None, :]
              + inj * m1_ref[...]).astype(bf)
        kf = (k * ks_ref[...] + kc_ref[...] + inj * m5_ref[...]).astype(bf)
        vb = v.astype(bf)
        qf_lo, qf_hi = qf[:half], qf[half:]
        kf_lo = kf[:half]
        s_lo = jax.lax.dot_general(qf_lo, kf_lo, (((1,), (1,)), ((), ())),
                                   preferred_element_type=jnp.float32) * scale
        p_lo, l_lo = masked_softmax(s_lo, causal_ll)
        a_lo = jnp.dot(p_lo, vb[:half],
                       preferred_element_type=jnp.float32) / l_lo
        s_hi = jax.lax.dot_general(qf_hi, kf, (((1,), (1,)), ((), ())),
                                   preferred_element_type=jnp.float32) * scale
        hi_mask = jnp.concatenate(
            [jnp.ones((half, half), jnp.bool_), causal_ll], axis=1)
        p_hi, l_hi = masked_softmax(s_hi, hi_mask)
        a_hi = jnp.dot(p_hi, vb, preferred_element_type=jnp.float32) / l_hi
        attn_h = jnp.concatenate([a_lo, a_hi], axis=0)
        part = jnp.dot(attn_h.astype(bf), wo_ref[h],
                       preferred_element_type=jnp.float32)
        acc = part if acc is None else acc + part

    xa = acc + bo_ref[...] + x
    ms2 = jnp.sum(xa * xa * mask_ref[...], axis=-1, keepdims=True) * inv_c
    xn2 = (xa * jax.lax.rsqrt(ms2 + RMS_EPS)).astype(bf)
    y = jnp.dot(xn2, wb_ref[...], preferred_element_type=jnp.float32)
    y = y + bb_ref[...]

    odim = wb_ref.shape[1] // 4
    c_i = odim // MV

    def to_bm(kk):
        tt = jnp.transpose(y[:, kk * odim:(kk + 1) * odim].astype(bf))
        return tt.reshape(MV, c_i, seq).astype(jnp.float32)

    lg, rg, rj = to_bm(0), to_bm(1), to_bm(3)
    lj = to_bm(2) * ps_ref[0, 0, 0]

    zparts = []
    for terms, a, bb2 in ((_GP_TERMS, lg, rg), (_JOIN_TERMS, lj, rj)):
        outs = []
        for n in range(16):
            nacc = None
            for (i, j, sgn) in terms[n]:
                t = a[i] * bb2[j]
                if sgn == -1.0:
                    t = -t
                elif sgn != 1.0:
                    t = t * sgn
                nacc = t if nacc is None else nacc + t
            outs.append(nacc if nacc is not None else jnp.zeros_like(a[0]))
        zp = jnp.stack(outs, axis=0)
        zp = jnp.transpose(zp.reshape(odim, seq).astype(bf))
        zparts.append(zp)
    z = jnp.concatenate(zparts, axis=1)

    t2 = jnp.dot(z, w1_ref[...], preferred_element_type=jnp.float32)
    t2 = t2 + b1_ref[...]
    z2 = t2[:, :cdim]
    gate = jax.nn.gelu(t2[:, cdim:], approximate=True)
    gated = (z2 * gate).astype(bf)
    out = jnp.dot(gated, w2_ref[...], preferred_element_type=jnp.float32)
    o_ref[0] = out + b2_ref[...] + xa


def _feature_constants(w_ipa, w_daa, c_h):
    heads = w_ipa.shape[0]
    cdim = c_h * MV
    nd = np.zeros((MV,), np.float32)
    nd[list(ND_LANES)] = 1.0
    tri = np.zeros((MV,), np.float32)
    tri[list(TRI_LANES)] = 1.0
    lane1 = np.zeros((MV,), np.float32)
    lane1[1] = 1.0
    lane5 = np.zeros((MV,), np.float32)
    lane5[5] = 1.0

    ndj = jnp.asarray(nd)
    trij = jnp.asarray(tri)
    qscale = (w_ipa[:, :, None] * ndj + 2.0 * w_daa[:, :, None] * trij)
    qscale = qscale.reshape(heads, cdim)
    qconst = (-w_daa[:, :, None] * jnp.asarray(lane5)).reshape(heads, cdim)
    kscale = np.tile(nd + tri, c_h).reshape(1, cdim)
    kconst = np.tile(lane1, c_h).reshape(1, cdim)

    pat1 = np.zeros((cdim, cdim), np.float32)
    pat5 = np.zeros((cdim, cdim), np.float32)
    for c in range(c_h):
        for t in TRI_LANES:
            pat1[c * MV + t, c * MV + 1] = 1.0
            pat5[c * MV + t, c * MV + 5] = 1.0
    col_w = (-w_daa[:, :, None] * jnp.ones((1, 1, MV))).reshape(heads, 1, cdim)
    top = jnp.asarray(pat1)[None] * col_w
    bot = jnp.broadcast_to(jnp.asarray(pat5)[None], top.shape)
    inj = jnp.concatenate([top, bot], axis=1).astype(jnp.bfloat16)
    m1 = np.tile(lane1, c_h).reshape(1, cdim)
    m5 = np.tile(lane5, c_h).reshape(1, cdim)
    return (qscale, qconst, jnp.asarray(kscale), jnp.asarray(kconst),
            inj, jnp.asarray(m1), jnp.asarray(m5))


def kernel(x, ref_input, qkv_w, qkv_b, attn_out_w, attn_out_b, bil_w, bil_b,
           bil_out_w, bil_out_b, mlp_out_w, mlp_out_b, w_ipa, w_daa,
           norm_mask):
    b, t, c_h, mv = x.shape
    assert mv == MV
    heads = w_ipa.shape[0]
    cdim = c_h * MV
    c_inter = bil_w.shape[1] // (4 * MV)
    inv_c = 1.0 / c_h
    scale = 1.0 / np.sqrt(c_h * 13)

    x3 = x.reshape(b, t, cdim)
    qs, qc, ks, kc, inj, m1, m5 = _feature_constants(w_ipa, w_daa, c_h)
    ref_ps = jnp.broadcast_to(ref_input[:, 0, 0, 15][:, None, None],
                              (b, 1, 128)).astype(jnp.float32)
    w1 = jnp.concatenate(
        [bil_out_w, jnp.repeat(bil_out_w[:, ::MV], MV, axis=1)], axis=1)
    b1 = jnp.concatenate(
        [bil_out_b, jnp.repeat(bil_out_b[:, ::MV], MV, axis=1)], axis=1)
    zdim = 2 * c_inter * MV
    odim = c_inter * MV
    bil_w_p = bil_w.reshape(cdim, 4, c_inter, MV)
    bil_w_p = jnp.transpose(bil_w_p, (0, 1, 3, 2)).reshape(cdim, 4 * odim)
    bil_b_p = bil_b.reshape(1, 4, c_inter, MV)
    bil_b_p = jnp.transpose(bil_b_p, (0, 1, 3, 2)).reshape(1, 4 * odim)
    w1_p = w1.reshape(2, c_inter, MV, 2 * cdim)
    w1_p = jnp.transpose(w1_p, (0, 2, 1, 3)).reshape(zdim, 2 * cdim)

    bf = jnp.bfloat16
    out = pl.pallas_call(
        functools.partial(_block_kernel, heads=heads, cdim=cdim, inv_c=inv_c,
                          seq=t, scale=scale),
        out_shape=jax.ShapeDtypeStruct((b, t, cdim), jnp.float32),
        grid=(b,),
        in_specs=[
            pl.BlockSpec((1, t, cdim), lambda i: (i, 0, 0)),
            pl.BlockSpec(qkv_w.shape, lambda i: (0, 0)),
            pl.BlockSpec(qkv_b.shape, lambda i: (0, 0)),
            pl.BlockSpec(norm_mask.shape, lambda i: (0, 0)),
            pl.BlockSpec((heads, cdim), lambda i: (0, 0)),
            pl.BlockSpec((heads, cdim), lambda i: (0, 0)),
            pl.BlockSpec((1, cdim), lambda i: (0, 0)),
            pl.BlockSpec((1, cdim), lambda i: (0, 0)),
            pl.BlockSpec((heads, 2 * cdim, cdim), lambda i: (0, 0, 0)),
            pl.BlockSpec((1, cdim), lambda i: (0, 0)),
            pl.BlockSpec((1, cdim), lambda i: (0, 0)),
            pl.BlockSpec((heads, cdim, cdim), lambda i: (0, 0, 0)),
            pl.BlockSpec(attn_out_b.shape, lambda i: (0, 0)),
            pl.BlockSpec(bil_w.shape, lambda i: (0, 0)),
            pl.BlockSpec(bil_b.shape, lambda i: (0, 0)),
            pl.BlockSpec((1, 1, 128), lambda i: (i, 0, 0)),
            pl.BlockSpec((zdim, 2 * cdim), lambda i: (0, 0)),
            pl.BlockSpec((1, 2 * cdim), lambda i: (0, 0)),
            pl.BlockSpec((cdim, cdim), lambda i: (0, 0)),
            pl.BlockSpec(mlp_out_b.shape, lambda i: (0, 0)),
        ],
        out_specs=pl.BlockSpec((1, t, cdim), lambda i: (i, 0, 0)),
        compiler_params=pltpu.CompilerParams(
            dimension_semantics=("parallel",),
            vmem_limit_bytes=100 * 1024 * 1024),
    )(x3, qkv_w.astype(bf), qkv_b, norm_mask, qs, qc, ks, kc, inj, m1, m5,
      attn_out_w.reshape(heads, cdim, cdim).astype(bf), attn_out_b,
      bil_w_p.astype(bf), bil_b_p, ref_ps, w1_p.astype(bf), b1,
      mlp_out_w.astype(bf), mlp_out_b)

    return out.reshape(b, t, c_h, MV)


# speedup vs baseline: 1.7256x; 1.1417x over previous
import functools

import numpy as np
import jax
import jax.numpy as jnp
from jax.experimental import pallas as pl
from jax.experimental.pallas import tpu as pltpu

MV = 16
RMS_EPS = 1e-6
ND_LANES = (0, 2, 3, 4, 8, 9, 10, 14)
TRI_LANES = (11, 12, 13)

_BLADES = [(), (0,), (1,), (2,), (3,), (0, 1), (0, 2), (0, 3), (1, 2), (1, 3),
           (2, 3), (0, 1, 2), (0, 1, 3), (0, 2, 3), (1, 2, 3), (0, 1, 2, 3)]
_B2I = {b: i for i, b in enumerate(_BLADES)}


def _perm_sign(seq):
    arr = list(seq)
    sgn = 1.0
    for a in range(1, len(arr)):
        b = a
        while b > 0 and arr[b - 1] > arr[b]:
            arr[b - 1], arr[b] = arr[b], arr[b - 1]
            sgn = -sgn
            b -= 1
    return sgn, arr


def _mul_blades(x, y):
    sgn, arr = _perm_sign(list(x) + list(y))
    out, i = [], 0
    while i < len(arr):
        if i + 1 < len(arr) and arr[i] == arr[i + 1]:
            if arr[i] == 0:
                return 0.0, ()
            i += 2
        else:
            out.append(arr[i])
            i += 1
    return sgn, tuple(out)


def _tables():
    gp = np.zeros((16, 16, 16), np.float32)
    wedge = np.zeros((16, 16, 16), np.float32)
    for i, a in enumerate(_BLADES):
        for j, b in enumerate(_BLADES):
            s, c = _mul_blades(a, b)
            if s:
                gp[i, j, _B2I[c]] = s
            if not (set(a) & set(b)):
                s2, arr = _perm_sign(list(a) + list(b))
                wedge[i, j, _B2I[tuple(arr)]] = s2
    dual = np.zeros((16, 16), np.float32)
    for i, bl in enumerate(_BLADES):
        comp = tuple(sorted(set((0, 1, 2, 3)) - set(bl)))
        s, _ = _perm_sign(list(bl) + list(comp))
        dual[_B2I[comp], i] = s
    join = np.einsum("mn,pqm,pi,qj->ijn", dual, wedge, dual, dual)
    return gp, join.astype(np.float32)


_GP_TBL, _JOIN_TBL = _tables()


def _term_list(tbl):
    out = [[] for _ in range(16)]
    for i, j, n in np.argwhere(tbl != 0.0):
        out[int(n)].append((int(i), int(j), float(tbl[i, j, n])))
    return out


_GP_TERMS = _term_list(_GP_TBL)
_JOIN_TERMS = _term_list(_JOIN_TBL)


def _block_kernel(x_ref, w_ref, b_ref, mask_ref, qs_ref, ks_ref, wtri_ref,
                  m1_ref, wo_ref, bo_ref, wb_ref,
                  bb_ref, ps_ref, w1_ref, b1_ref, w2_ref, b2_ref, o_ref, *,
                  heads, cdim, inv_c, seq, scale):
    bf = jnp.bfloat16
    x = x_ref[0]
    ms = jnp.sum(x * x * mask_ref[...], axis=-1, keepdims=True) * inv_c
    xn = (x * jax.lax.rsqrt(ms + RMS_EPS)).astype(bf)
    qkv = jnp.dot(xn, w_ref[...], preferred_element_type=jnp.float32)
    qkv = qkv + b_ref[...]

    half = seq // 2
    row = jax.lax.broadcasted_iota(jnp.int32, (half, half), 0)
    col = jax.lax.broadcasted_iota(jnp.int32, (half, half), 1)
    causal_ll = col <= row

    def masked_softmax(s, causal):
        s = jnp.where(causal, s, -1e30)
        m = jnp.max(s, axis=-1, keepdims=True)
        p = jnp.exp(s - m)
        l = jnp.sum(p, axis=-1, keepdims=True)
        return p.astype(bf), l

    acc = None
    for h in range(heads):
        q = qkv[:, h * cdim:(h + 1) * cdim]
        k = qkv[:, (heads + h) * cdim:(heads + h + 1) * cdim]
        v = qkv[:, (2 * heads + h) * cdim:(2 * heads + h + 1) * cdim]
        cv = jnp.sum((k * k) * wtri_ref[h][None, :], axis=-1, keepdims=True)
        qf = (q * qs_ref[h][None, :] + m1_ref[...]).astype(bf)
        kf = (k * ks_ref[...] + cv * m1_ref[...]).astype(bf)
        vb = v.astype(bf)
        qf_lo, qf_hi = qf[:half], qf[half:]
        kf_lo = kf[:half]
        s_lo = jax.lax.dot_general(qf_lo, kf_lo, (((1,), (1,)), ((), ())),
                                   preferred_element_type=jnp.float32) * scale
        p_lo, l_lo = masked_softmax(s_lo, causal_ll)
        a_lo = jnp.dot(p_lo, vb[:half],
                       preferred_element_type=jnp.float32) / l_lo
        s_hi = jax.lax.dot_general(qf_hi, kf, (((1,), (1,)), ((), ())),
                                   preferred_element_type=jnp.float32) * scale
        hi_mask = jnp.concatenate(
            [jnp.ones((half, half), jnp.bool_), causal_ll], axis=1)
        p_hi, l_hi = masked_softmax(s_hi, hi_mask)
        a_hi = jnp.dot(p_hi, vb, preferred_element_type=jnp.float32) / l_hi
        attn_h = jnp.concatenate([a_lo, a_hi], axis=0)
        part = jnp.dot(attn_h.astype(bf), wo_ref[h],
                       preferred_element_type=jnp.float32)
        acc = part if acc is None else acc + part

    xa = acc + bo_ref[...] + x
    ms2 = jnp.sum(xa * xa * mask_ref[...], axis=-1, keepdims=True) * inv_c
    xn2 = (xa * jax.lax.rsqrt(ms2 + RMS_EPS)).astype(bf)
    y = jnp.dot(xn2, wb_ref[...], preferred_element_type=jnp.float32)
    y = y + bb_ref[...]

    odim = wb_ref.shape[1] // 4
    c_i = odim // MV

    def to_bm(kk):
        tt = jnp.transpose(y[:, kk * odim:(kk + 1) * odim].astype(bf))
        return tt.reshape(MV, c_i, seq).astype(jnp.float32)

    lg, rg, rj = to_bm(0), to_bm(1), to_bm(3)
    lj = to_bm(2) * ps_ref[0, 0, 0]

    zparts = []
    for terms, a, bb2 in ((_GP_TERMS, lg, rg), (_JOIN_TERMS, lj, rj)):
        outs = []
        for n in range(16):
            nacc = None
            for (i, j, sgn) in terms[n]:
                t = a[i] * bb2[j]
                if sgn == -1.0:
                    t = -t
                elif sgn != 1.0:
                    t = t * sgn
                nacc = t if nacc is None else nacc + t
            outs.append(nacc if nacc is not None else jnp.zeros_like(a[0]))
        zp = jnp.stack(outs, axis=0)
        zp = jnp.transpose(zp.reshape(odim, seq).astype(bf))
        zparts.append(zp)
    z = jnp.concatenate(zparts, axis=1)

    t2 = jnp.dot(z, w1_ref[...], preferred_element_type=jnp.float32)
    t2 = t2 + b1_ref[...]
    z2 = t2[:, :cdim]
    gate = jax.nn.gelu(t2[:, cdim:], approximate=True)
    gated = (z2 * gate).astype(bf)
    out = jnp.dot(gated, w2_ref[...], preferred_element_type=jnp.float32)
    o_ref[0] = out + b2_ref[...] + xa


def _feature_constants(w_ipa, w_daa, c_h):
    heads = w_ipa.shape[0]
    cdim = c_h * MV
    nd = np.zeros((MV,), np.float32)
    nd[list(ND_LANES)] = 1.0
    tri = np.zeros((MV,), np.float32)
    tri[list(TRI_LANES)] = 1.0

    qscale = (w_ipa[:, :, None] * jnp.asarray(nd)
              + 2.0 * w_daa[:, :, None] * jnp.asarray(tri))
    qscale = qscale.reshape(heads, cdim)
    kscale = np.tile(nd + tri, c_h).reshape(1, cdim)
    wtri = (-w_daa[:, :, None] * jnp.asarray(tri)).reshape(heads, cdim)
    m1 = np.zeros((1, cdim), np.float32)
    m1[0, 1] = 1.0
    return qscale, jnp.asarray(kscale), wtri, jnp.asarray(m1)


def kernel(x, ref_input, qkv_w, qkv_b, attn_out_w, attn_out_b, bil_w, bil_b,
           bil_out_w, bil_out_b, mlp_out_w, mlp_out_b, w_ipa, w_daa,
           norm_mask):
    b, t, c_h, mv = x.shape
    assert mv == MV
    heads = w_ipa.shape[0]
    cdim = c_h * MV
    c_inter = bil_w.shape[1] // (4 * MV)
    inv_c = 1.0 / c_h
    scale = 1.0 / np.sqrt(c_h * 13)

    x3 = x.reshape(b, t, cdim)
    qs, ks, wtri, m1 = _feature_constants(w_ipa, w_daa, c_h)
    ref_ps = jnp.broadcast_to(ref_input[:, 0, 0, 15][:, None, None],
                              (b, 1, 128)).astype(jnp.float32)
    w1 = jnp.concatenate(
        [bil_out_w, jnp.repeat(bil_out_w[:, ::MV], MV, axis=1)], axis=1)
    b1 = jnp.concatenate(
        [bil_out_b, jnp.repeat(bil_out_b[:, ::MV], MV, axis=1)], axis=1)
    zdim = 2 * c_inter * MV
    odim = c_inter * MV
    bil_w_p = bil_w.reshape(cdim, 4, c_inter, MV)
    bil_w_p = jnp.transpose(bil_w_p, (0, 1, 3, 2)).reshape(cdim, 4 * odim)
    bil_b_p = bil_b.reshape(1, 4, c_inter, MV)
    bil_b_p = jnp.transpose(bil_b_p, (0, 1, 3, 2)).reshape(1, 4 * odim)
    w1_p = w1.reshape(2, c_inter, MV, 2 * cdim)
    w1_p = jnp.transpose(w1_p, (0, 2, 1, 3)).reshape(zdim, 2 * cdim)

    bf = jnp.bfloat16
    out = pl.pallas_call(
        functools.partial(_block_kernel, heads=heads, cdim=cdim, inv_c=inv_c,
                          seq=t, scale=scale),
        out_shape=jax.ShapeDtypeStruct((b, t, cdim), jnp.float32),
        grid=(b,),
        in_specs=[
            pl.BlockSpec((1, t, cdim), lambda i: (i, 0, 0)),
            pl.BlockSpec(qkv_w.shape, lambda i: (0, 0)),
            pl.BlockSpec(qkv_b.shape, lambda i: (0, 0)),
            pl.BlockSpec(norm_mask.shape, lambda i: (0, 0)),
            pl.BlockSpec((heads, cdim), lambda i: (0, 0)),
            pl.BlockSpec((1, cdim), lambda i: (0, 0)),
            pl.BlockSpec((heads, cdim), lambda i: (0, 0)),
            pl.BlockSpec((1, cdim), lambda i: (0, 0)),
            pl.BlockSpec((heads, cdim, cdim), lambda i: (0, 0, 0)),
            pl.BlockSpec(attn_out_b.shape, lambda i: (0, 0)),
            pl.BlockSpec(bil_w.shape, lambda i: (0, 0)),
            pl.BlockSpec(bil_b.shape, lambda i: (0, 0)),
            pl.BlockSpec((1, 1, 128), lambda i: (i, 0, 0)),
            pl.BlockSpec((zdim, 2 * cdim), lambda i: (0, 0)),
            pl.BlockSpec((1, 2 * cdim), lambda i: (0, 0)),
            pl.BlockSpec((cdim, cdim), lambda i: (0, 0)),
            pl.BlockSpec(mlp_out_b.shape, lambda i: (0, 0)),
        ],
        out_specs=pl.BlockSpec((1, t, cdim), lambda i: (i, 0, 0)),
        compiler_params=pltpu.CompilerParams(
            dimension_semantics=("parallel",),
            vmem_limit_bytes=100 * 1024 * 1024),
    )(x3, qkv_w.astype(bf), qkv_b, norm_mask, qs, ks, wtri, m1,
      attn_out_w.reshape(heads, cdim, cdim).astype(bf), attn_out_b,
      bil_w_p.astype(bf), bil_b_p, ref_ps, w1_p.astype(bf), b1,
      mlp_out_w.astype(bf), mlp_out_b)

    return out.reshape(b, t, c_h, MV)


# speedup vs baseline: 1.7387x; 1.0076x over previous
import functools

import numpy as np
import jax
import jax.numpy as jnp
from jax.experimental import pallas as pl
from jax.experimental.pallas import tpu as pltpu

MV = 16
RMS_EPS = 1e-6
ND_LANES = (0, 2, 3, 4, 8, 9, 10, 14)
TRI_LANES = (11, 12, 13)

_BLADES = [(), (0,), (1,), (2,), (3,), (0, 1), (0, 2), (0, 3), (1, 2), (1, 3),
           (2, 3), (0, 1, 2), (0, 1, 3), (0, 2, 3), (1, 2, 3), (0, 1, 2, 3)]
_B2I = {b: i for i, b in enumerate(_BLADES)}


def _perm_sign(seq):
    arr = list(seq)
    sgn = 1.0
    for a in range(1, len(arr)):
        b = a
        while b > 0 and arr[b - 1] > arr[b]:
            arr[b - 1], arr[b] = arr[b], arr[b - 1]
            sgn = -sgn
            b -= 1
    return sgn, arr


def _mul_blades(x, y):
    sgn, arr = _perm_sign(list(x) + list(y))
    out, i = [], 0
    while i < len(arr):
        if i + 1 < len(arr) and arr[i] == arr[i + 1]:
            if arr[i] == 0:
                return 0.0, ()
            i += 2
        else:
            out.append(arr[i])
            i += 1
    return sgn, tuple(out)


def _tables():
    gp = np.zeros((16, 16, 16), np.float32)
    wedge = np.zeros((16, 16, 16), np.float32)
    for i, a in enumerate(_BLADES):
        for j, b in enumerate(_BLADES):
            s, c = _mul_blades(a, b)
            if s:
                gp[i, j, _B2I[c]] = s
            if not (set(a) & set(b)):
                s2, arr = _perm_sign(list(a) + list(b))
                wedge[i, j, _B2I[tuple(arr)]] = s2
    dual = np.zeros((16, 16), np.float32)
    for i, bl in enumerate(_BLADES):
        comp = tuple(sorted(set((0, 1, 2, 3)) - set(bl)))
        s, _ = _perm_sign(list(bl) + list(comp))
        dual[_B2I[comp], i] = s
    join = np.einsum("mn,pqm,pi,qj->ijn", dual, wedge, dual, dual)
    return gp, join.astype(np.float32)


_GP_TBL, _JOIN_TBL = _tables()


def _term_list(tbl):
    out = [[] for _ in range(16)]
    for i, j, n in np.argwhere(tbl != 0.0):
        out[int(n)].append((int(i), int(j), float(tbl[i, j, n])))
    return out


_GP_TERMS = _term_list(_GP_TBL)
_JOIN_TERMS = _term_list(_JOIN_TBL)


def _block_kernel(x_ref, w_ref, b_ref, mask_ref, qs_ref, ks_ref, wtri_ref,
                  m1_ref, wo_ref, bo_ref, wb_ref,
                  bb_ref, ps_ref, w1_ref, b1_ref, w2_ref, b2_ref, o_ref, *,
                  heads, cdim, inv_c, seq, scale):
    bf = jnp.bfloat16
    x = x_ref[0]
    ms = jnp.sum(x * x * mask_ref[...], axis=-1, keepdims=True) * inv_c
    xn = (x * jax.lax.rsqrt(ms + RMS_EPS)).astype(bf)
    qkv = jnp.dot(xn, w_ref[...], preferred_element_type=jnp.float32)
    qkv = qkv + b_ref[...]

    half = seq // 2
    row = jax.lax.broadcasted_iota(jnp.int32, (half, half), 0)
    col = jax.lax.broadcasted_iota(jnp.int32, (half, half), 1)
    causal_ll = col <= row

    def masked_softmax(s, causal):
        s = jnp.where(causal, s, -1e30)
        m = jnp.max(s, axis=-1, keepdims=True)
        p = jnp.exp(s - m)
        l = jnp.sum(p, axis=-1, keepdims=True)
        return p.astype(bf), l

    parts = []
    for h in range(heads):
        q = qkv[:, h * cdim:(h + 1) * cdim]
        k = qkv[:, (heads + h) * cdim:(heads + h + 1) * cdim]
        v = qkv[:, (2 * heads + h) * cdim:(2 * heads + h + 1) * cdim]
        cv = jnp.sum((k * k) * wtri_ref[h][None, :], axis=-1, keepdims=True)
        qf = (q * qs_ref[h][None, :] + m1_ref[...]).astype(bf)
        kf = (k * ks_ref[...] + cv * m1_ref[...]).astype(bf)
        vb = v.astype(bf)
        qf_lo, qf_hi = qf[:half], qf[half:]
        kf_lo = kf[:half]
        s_lo = jax.lax.dot_general(qf_lo, kf_lo, (((1,), (1,)), ((), ())),
                                   preferred_element_type=jnp.float32) * scale
        p_lo, l_lo = masked_softmax(s_lo, causal_ll)
        a_lo = jnp.dot(p_lo, vb[:half],
                       preferred_element_type=jnp.float32) * (1.0 / l_lo)
        s_hi = jax.lax.dot_general(qf_hi, kf, (((1,), (1,)), ((), ())),
                                   preferred_element_type=jnp.float32) * scale
        hi_mask = jnp.concatenate(
            [jnp.ones((half, half), jnp.bool_), causal_ll], axis=1)
        p_hi, l_hi = masked_softmax(s_hi, hi_mask)
        a_hi = jnp.dot(p_hi, vb,
                       preferred_element_type=jnp.float32) * (1.0 / l_hi)
        attn_h = jnp.concatenate([a_lo, a_hi], axis=0)
        parts.append(attn_h.astype(bf))

    attn_cat = jnp.concatenate(parts, axis=1)
    acc = jnp.dot(attn_cat, wo_ref[...], preferred_element_type=jnp.float32)
    xa = acc + bo_ref[...] + x
    ms2 = jnp.sum(xa * xa * mask_ref[...], axis=-1, keepdims=True) * inv_c
    xn2 = (xa * jax.lax.rsqrt(ms2 + RMS_EPS)).astype(bf)
    y = jnp.dot(xn2, wb_ref[...], preferred_element_type=jnp.float32)
    y = y + bb_ref[...]

    odim = wb_ref.shape[1] // 4
    c_i = odim // MV

    def to_bm(kk):
        tt = jnp.transpose(y[:, kk * odim:(kk + 1) * odim].astype(bf))
        return tt.reshape(MV, c_i, seq).astype(jnp.float32)

    lg, rg, rj = to_bm(0), to_bm(1), to_bm(3)
    lj = to_bm(2) * ps_ref[0, 0, 0]

    zparts = []
    for terms, a, bb2 in ((_GP_TERMS, lg, rg), (_JOIN_TERMS, lj, rj)):
        outs = []
        for n in range(16):
            nacc = None
            for (i, j, sgn) in terms[n]:
                t = a[i] * bb2[j]
                if sgn == -1.0:
                    t = -t
                elif sgn != 1.0:
                    t = t * sgn
                nacc = t if nacc is None else nacc + t
            outs.append(nacc if nacc is not None else jnp.zeros_like(a[0]))
        zp = jnp.stack(outs, axis=0)
        zp = jnp.transpose(zp.reshape(odim, seq).astype(bf))
        zparts.append(zp)
    z = jnp.concatenate(zparts, axis=1)

    t2 = jnp.dot(z, w1_ref[...], preferred_element_type=jnp.float32)
    t2 = t2 + b1_ref[...]
    z2 = t2[:, :cdim]
    gate = jax.nn.gelu(t2[:, cdim:], approximate=True)
    gated = (z2 * gate).astype(bf)
    out = jnp.dot(gated, w2_ref[...], preferred_element_type=jnp.float32)
    o_ref[0] = out + b2_ref[...] + xa


def _feature_constants(w_ipa, w_daa, c_h):
    heads = w_ipa.shape[0]
    cdim = c_h * MV
    nd = np.zeros((MV,), np.float32)
    nd[list(ND_LANES)] = 1.0
    tri = np.zeros((MV,), np.float32)
    tri[list(TRI_LANES)] = 1.0

    qscale = (w_ipa[:, :, None] * jnp.asarray(nd)
              + 2.0 * w_daa[:, :, None] * jnp.asarray(tri))
    qscale = qscale.reshape(heads, cdim)
    kscale = np.tile(nd + tri, c_h).reshape(1, cdim)
    wtri = (-w_daa[:, :, None] * jnp.asarray(tri)).reshape(heads, cdim)
    m1 = np.zeros((1, cdim), np.float32)
    m1[0, 1] = 1.0
    return qscale, jnp.asarray(kscale), wtri, jnp.asarray(m1)


def kernel(x, ref_input, qkv_w, qkv_b, attn_out_w, attn_out_b, bil_w, bil_b,
           bil_out_w, bil_out_b, mlp_out_w, mlp_out_b, w_ipa, w_daa,
           norm_mask):
    b, t, c_h, mv = x.shape
    assert mv == MV
    heads = w_ipa.shape[0]
    cdim = c_h * MV
    c_inter = bil_w.shape[1] // (4 * MV)
    inv_c = 1.0 / c_h
    scale = 1.0 / np.sqrt(c_h * 13)

    x3 = x.reshape(b, t, cdim)
    qs, ks, wtri, m1 = _feature_constants(w_ipa, w_daa, c_h)
    ref_ps = jnp.broadcast_to(ref_input[:, 0, 0, 15][:, None, None],
                              (b, 1, 128)).astype(jnp.float32)
    w1 = jnp.concatenate(
        [bil_out_w, jnp.repeat(bil_out_w[:, ::MV], MV, axis=1)], axis=1)
    b1 = jnp.concatenate(
        [bil_out_b, jnp.repeat(bil_out_b[:, ::MV], MV, axis=1)], axis=1)
    zdim = 2 * c_inter * MV
    odim = c_inter * MV
    bil_w_p = bil_w.reshape(cdim, 4, c_inter, MV)
    bil_w_p = jnp.transpose(bil_w_p, (0, 1, 3, 2)).reshape(cdim, 4 * odim)
    bil_b_p = bil_b.reshape(1, 4, c_inter, MV)
    bil_b_p = jnp.transpose(bil_b_p, (0, 1, 3, 2)).reshape(1, 4 * odim)
    w1_p = w1.reshape(2, c_inter, MV, 2 * cdim)
    w1_p = jnp.transpose(w1_p, (0, 2, 1, 3)).reshape(zdim, 2 * cdim)

    bf = jnp.bfloat16
    out = pl.pallas_call(
        functools.partial(_block_kernel, heads=heads, cdim=cdim, inv_c=inv_c,
                          seq=t, scale=scale),
        out_shape=jax.ShapeDtypeStruct((b, t, cdim), jnp.float32),
        grid=(b,),
        in_specs=[
            pl.BlockSpec((1, t, cdim), lambda i: (i, 0, 0)),
            pl.BlockSpec(qkv_w.shape, lambda i: (0, 0)),
            pl.BlockSpec(qkv_b.shape, lambda i: (0, 0)),
            pl.BlockSpec(norm_mask.shape, lambda i: (0, 0)),
            pl.BlockSpec((heads, cdim), lambda i: (0, 0)),
            pl.BlockSpec((1, cdim), lambda i: (0, 0)),
            pl.BlockSpec((heads, cdim), lambda i: (0, 0)),
            pl.BlockSpec((1, cdim), lambda i: (0, 0)),
            pl.BlockSpec((heads * cdim, cdim), lambda i: (0, 0)),
            pl.BlockSpec(attn_out_b.shape, lambda i: (0, 0)),
            pl.BlockSpec(bil_w.shape, lambda i: (0, 0)),
            pl.BlockSpec(bil_b.shape, lambda i: (0, 0)),
            pl.BlockSpec((1, 1, 128), lambda i: (i, 0, 0)),
            pl.BlockSpec((zdim, 2 * cdim), lambda i: (0, 0)),
            pl.BlockSpec((1, 2 * cdim), lambda i: (0, 0)),
            pl.BlockSpec((cdim, cdim), lambda i: (0, 0)),
            pl.BlockSpec(mlp_out_b.shape, lambda i: (0, 0)),
        ],
        out_specs=pl.BlockSpec((1, t, cdim), lambda i: (i, 0, 0)),
        compiler_params=pltpu.CompilerParams(
            dimension_semantics=("parallel",),
            vmem_limit_bytes=100 * 1024 * 1024),
    )(x3, qkv_w.astype(bf), qkv_b, norm_mask, qs, ks, wtri, m1,
      attn_out_w.astype(bf), attn_out_b,
      bil_w_p.astype(bf), bil_b_p, ref_ps, w1_p.astype(bf), b1,
      mlp_out_w.astype(bf), mlp_out_b)

    return out.reshape(b, t, c_h, MV)


# speedup vs baseline: 1.9157x; 1.1018x over previous
import functools

import numpy as np
import jax
import jax.numpy as jnp
from jax.experimental import pallas as pl
from jax.experimental.pallas import tpu as pltpu

MV = 16
RMS_EPS = 1e-6
ND_LANES = (0, 2, 3, 4, 8, 9, 10, 14)
TRI_LANES = (11, 12, 13)

_BLADES = [(), (0,), (1,), (2,), (3,), (0, 1), (0, 2), (0, 3), (1, 2), (1, 3),
           (2, 3), (0, 1, 2), (0, 1, 3), (0, 2, 3), (1, 2, 3), (0, 1, 2, 3)]
_B2I = {b: i for i, b in enumerate(_BLADES)}


def _perm_sign(seq):
    arr = list(seq)
    sgn = 1.0
    for a in range(1, len(arr)):
        b = a
        while b > 0 and arr[b - 1] > arr[b]:
            arr[b - 1], arr[b] = arr[b], arr[b - 1]
            sgn = -sgn
            b -= 1
    return sgn, arr


def _mul_blades(x, y):
    sgn, arr = _perm_sign(list(x) + list(y))
    out, i = [], 0
    while i < len(arr):
        if i + 1 < len(arr) and arr[i] == arr[i + 1]:
            if arr[i] == 0:
                return 0.0, ()
            i += 2
        else:
            out.append(arr[i])
            i += 1
    return sgn, tuple(out)


def _tables():
    gp = np.zeros((16, 16, 16), np.float32)
    wedge = np.zeros((16, 16, 16), np.float32)
    for i, a in enumerate(_BLADES):
        for j, b in enumerate(_BLADES):
            s, c = _mul_blades(a, b)
            if s:
                gp[i, j, _B2I[c]] = s
            if not (set(a) & set(b)):
                s2, arr = _perm_sign(list(a) + list(b))
                wedge[i, j, _B2I[tuple(arr)]] = s2
    dual = np.zeros((16, 16), np.float32)
    for i, bl in enumerate(_BLADES):
        comp = tuple(sorted(set((0, 1, 2, 3)) - set(bl)))
        s, _ = _perm_sign(list(bl) + list(comp))
        dual[_B2I[comp], i] = s
    join = np.einsum("mn,pqm,pi,qj->ijn", dual, wedge, dual, dual)
    return gp, join.astype(np.float32)


_GP_TBL, _JOIN_TBL = _tables()


def _term_list(tbl):
    out = [[] for _ in range(16)]
    for i, j, n in np.argwhere(tbl != 0.0):
        out[int(n)].append((int(i), int(j), float(tbl[i, j, n])))
    return out


_GP_TERMS = _term_list(_GP_TBL)
_JOIN_TERMS = _term_list(_JOIN_TBL)


def _block_kernel(x_ref, w_ref, b_ref, mask_ref, qs_ref, ks_ref, wtri_ref,
                  m1_ref, wo_ref, bo_ref, wb_ref,
                  bb_ref, ps_ref, w1_ref, b1_ref, wsc_ref, bsc_ref, bc_ref,
                  w2_ref, b2_ref, o_ref, *,
                  heads, cdim, inv_c, seq, scale):
    bf = jnp.bfloat16
    x = x_ref[0]
    ms = jnp.sum(x * x * mask_ref[...], axis=-1, keepdims=True) * inv_c
    xn = (x * jax.lax.rsqrt(ms + RMS_EPS)).astype(bf)
    qkv = jnp.dot(xn, w_ref[...], preferred_element_type=jnp.float32)
    qkv = qkv + b_ref[...]

    half = seq // 2
    row = jax.lax.broadcasted_iota(jnp.int32, (half, half), 0)
    col = jax.lax.broadcasted_iota(jnp.int32, (half, half), 1)
    causal_ll = col <= row

    def masked_softmax(s, causal):
        s = jnp.where(causal, s, -1e30)
        m = jnp.max(s, axis=-1, keepdims=True)
        p = jnp.exp(s - m)
        l = jnp.sum(p, axis=-1, keepdims=True)
        return p.astype(bf), l

    parts = []
    for h in range(heads):
        q = qkv[:, h * cdim:(h + 1) * cdim]
        k = qkv[:, (heads + h) * cdim:(heads + h + 1) * cdim]
        v = qkv[:, (2 * heads + h) * cdim:(2 * heads + h + 1) * cdim]
        cv = jnp.sum((k * k) * wtri_ref[h][None, :], axis=-1, keepdims=True)
        qf = (q * qs_ref[h][None, :] + m1_ref[...]).astype(bf)
        kf = (k * ks_ref[...] + cv * m1_ref[...]).astype(bf)
        vb = v.astype(bf)
        qf_lo, qf_hi = qf[:half], qf[half:]
        kf_lo = kf[:half]
        s_lo = jax.lax.dot_general(qf_lo, kf_lo, (((1,), (1,)), ((), ())),
                                   preferred_element_type=jnp.float32) * scale
        p_lo, l_lo = masked_softmax(s_lo, causal_ll)
        a_lo = jnp.dot(p_lo, vb[:half],
                       preferred_element_type=jnp.float32) * (1.0 / l_lo)
        s_hi = jax.lax.dot_general(qf_hi, kf, (((1,), (1,)), ((), ())),
                                   preferred_element_type=jnp.float32) * scale
        hi_mask = jnp.concatenate(
            [jnp.ones((half, half), jnp.bool_), causal_ll], axis=1)
        p_hi, l_hi = masked_softmax(s_hi, hi_mask)
        a_hi = jnp.dot(p_hi, vb,
                       preferred_element_type=jnp.float32) * (1.0 / l_hi)
        attn_h = jnp.concatenate([a_lo, a_hi], axis=0)
        parts.append(attn_h.astype(bf))

    attn_cat = jnp.concatenate(parts, axis=1)
    acc = jnp.dot(attn_cat, wo_ref[...], preferred_element_type=jnp.float32)
    xa = acc + bo_ref[...] + x
    ms2 = jnp.sum(xa * xa * mask_ref[...], axis=-1, keepdims=True) * inv_c
    xn2 = (xa * jax.lax.rsqrt(ms2 + RMS_EPS)).astype(bf)
    y = jnp.dot(xn2, wb_ref[...], preferred_element_type=jnp.float32)
    y = y + bb_ref[...]

    odim = wb_ref.shape[1] // 4
    c_i = odim // MV

    def to_bm(kk):
        tt = jnp.transpose(y[:, kk * odim:(kk + 1) * odim].astype(bf))
        return tt.reshape(MV, c_i, seq).astype(jnp.float32)

    lg, rg, rj = to_bm(0), to_bm(1), to_bm(3)
    lj = to_bm(2) * ps_ref[0, 0, 0]

    zparts = []
    for terms, a, bb2 in ((_GP_TERMS, lg, rg), (_JOIN_TERMS, lj, rj)):
        outs = []
        for n in range(16):
            nacc = None
            for (i, j, sgn) in terms[n]:
                t = a[i] * bb2[j]
                if sgn == -1.0:
                    t = -t
                elif sgn != 1.0:
                    t = t * sgn
                nacc = t if nacc is None else nacc + t
            outs.append(nacc if nacc is not None else jnp.zeros_like(a[0]))
        zp = jnp.stack(outs, axis=0)
        zp = jnp.transpose(zp.reshape(odim, seq).astype(bf))
        zparts.append(zp)
    z = jnp.concatenate(zparts, axis=1)

    z2 = jnp.dot(z, w1_ref[...], preferred_element_type=jnp.float32)
    z2 = z2 + b1_ref[...]
    g32 = jnp.dot(z, wsc_ref[...], preferred_element_type=jnp.float32)
    g32 = jax.nn.gelu(g32 + bsc_ref[...], approximate=True)
    gate = jnp.dot(g32.astype(bf), bc_ref[...],
                   preferred_element_type=jnp.float32)
    gated = (z2 * gate).astype(bf)
    out = jnp.dot(gated, w2_ref[...], preferred_element_type=jnp.float32)
    o_ref[0] = out + b2_ref[...] + xa


def _feature_constants(w_ipa, w_daa, c_h):
    heads = w_ipa.shape[0]
    cdim = c_h * MV
    nd = np.zeros((MV,), np.float32)
    nd[list(ND_LANES)] = 1.0
    tri = np.zeros((MV,), np.float32)
    tri[list(TRI_LANES)] = 1.0

    qscale = (w_ipa[:, :, None] * jnp.asarray(nd)
              + 2.0 * w_daa[:, :, None] * jnp.asarray(tri))
    qscale = qscale.reshape(heads, cdim)
    kscale = np.tile(nd + tri, c_h).reshape(1, cdim)
    wtri = (-w_daa[:, :, None] * jnp.asarray(tri)).reshape(heads, cdim)
    m1 = np.zeros((1, cdim), np.float32)
    m1[0, 1] = 1.0
    return qscale, jnp.asarray(kscale), wtri, jnp.asarray(m1)


def kernel(x, ref_input, qkv_w, qkv_b, attn_out_w, attn_out_b, bil_w, bil_b,
           bil_out_w, bil_out_b, mlp_out_w, mlp_out_b, w_ipa, w_daa,
           norm_mask):
    b, t, c_h, mv = x.shape
    assert mv == MV
    heads = w_ipa.shape[0]
    cdim = c_h * MV
    c_inter = bil_w.shape[1] // (4 * MV)
    inv_c = 1.0 / c_h
    scale = 1.0 / np.sqrt(c_h * 13)

    x3 = x.reshape(b, t, cdim)
    qs, ks, wtri, m1 = _feature_constants(w_ipa, w_daa, c_h)
    ref_ps = jnp.broadcast_to(ref_input[:, 0, 0, 15][:, None, None],
                              (b, 1, 128)).astype(jnp.float32)
    zdim = 2 * c_inter * MV
    wsc = bil_out_w[:, ::MV]
    bsc = bil_out_b[:, ::MV]
    bcast = np.zeros((c_h, cdim), np.float32)
    for c in range(c_h):
        bcast[c, c * MV:(c + 1) * MV] = 1.0
    bcast = jnp.asarray(bcast)
    odim = c_inter * MV
    bil_w_p = bil_w.reshape(cdim, 4, c_inter, MV)
    bil_w_p = jnp.transpose(bil_w_p, (0, 1, 3, 2)).reshape(cdim, 4 * odim)
    bil_b_p = bil_b.reshape(1, 4, c_inter, MV)
    bil_b_p = jnp.transpose(bil_b_p, (0, 1, 3, 2)).reshape(1, 4 * odim)
    w1_p = bil_out_w.reshape(2, c_inter, MV, cdim)
    w1_p = jnp.transpose(w1_p, (0, 2, 1, 3)).reshape(zdim, cdim)
    wsc_p = wsc.reshape(2, c_inter, MV, c_h)
    wsc_p = jnp.transpose(wsc_p, (0, 2, 1, 3)).reshape(zdim, c_h)

    bf = jnp.bfloat16
    out = pl.pallas_call(
        functools.partial(_block_kernel, heads=heads, cdim=cdim, inv_c=inv_c,
                          seq=t, scale=scale),
        out_shape=jax.ShapeDtypeStruct((b, t, cdim), jnp.float32),
        grid=(b,),
        in_specs=[
            pl.BlockSpec((1, t, cdim), lambda i: (i, 0, 0)),
            pl.BlockSpec(qkv_w.shape, lambda i: (0, 0)),
            pl.BlockSpec(qkv_b.shape, lambda i: (0, 0)),
            pl.BlockSpec(norm_mask.shape, lambda i: (0, 0)),
            pl.BlockSpec((heads, cdim), lambda i: (0, 0)),
            pl.BlockSpec((1, cdim), lambda i: (0, 0)),
            pl.BlockSpec((heads, cdim), lambda i: (0, 0)),
            pl.BlockSpec((1, cdim), lambda i: (0, 0)),
            pl.BlockSpec((heads * cdim, cdim), lambda i: (0, 0)),
            pl.BlockSpec(attn_out_b.shape, lambda i: (0, 0)),
            pl.BlockSpec(bil_w.shape, lambda i: (0, 0)),
            pl.BlockSpec(bil_b.shape, lambda i: (0, 0)),
            pl.BlockSpec((1, 1, 128), lambda i: (i, 0, 0)),
            pl.BlockSpec((zdim, cdim), lambda i: (0, 0)),
            pl.BlockSpec((1, cdim), lambda i: (0, 0)),
            pl.BlockSpec((zdim, c_h), lambda i: (0, 0)),
            pl.BlockSpec((1, c_h), lambda i: (0, 0)),
            pl.BlockSpec((c_h, cdim), lambda i: (0, 0)),
            pl.BlockSpec((cdim, cdim), lambda i: (0, 0)),
            pl.BlockSpec(mlp_out_b.shape, lambda i: (0, 0)),
        ],
        out_specs=pl.BlockSpec((1, t, cdim), lambda i: (i, 0, 0)),
        compiler_params=pltpu.CompilerParams(
            dimension_semantics=("parallel",),
            vmem_limit_bytes=100 * 1024 * 1024),
    )(x3, qkv_w.astype(bf), qkv_b, norm_mask, qs, ks, wtri, m1,
      attn_out_w.astype(bf), attn_out_b,
      bil_w_p.astype(bf), bil_b_p, ref_ps, w1_p.astype(bf), bil_out_b,
      wsc_p.astype(bf), bsc, bcast.astype(bf),
      mlp_out_w.astype(bf), mlp_out_b)

    return out.reshape(b, t, c_h, MV)


# speedup vs baseline: 2.0171x; 1.0529x over previous
import functools

import numpy as np
import jax
import jax.numpy as jnp
from jax.experimental import pallas as pl
from jax.experimental.pallas import tpu as pltpu

MV = 16
RMS_EPS = 1e-6
ND_LANES = (0, 2, 3, 4, 8, 9, 10, 14)
TRI_LANES = (11, 12, 13)

_BLADES = [(), (0,), (1,), (2,), (3,), (0, 1), (0, 2), (0, 3), (1, 2), (1, 3),
           (2, 3), (0, 1, 2), (0, 1, 3), (0, 2, 3), (1, 2, 3), (0, 1, 2, 3)]
_B2I = {b: i for i, b in enumerate(_BLADES)}


def _perm_sign(seq):
    arr = list(seq)
    sgn = 1.0
    for a in range(1, len(arr)):
        b = a
        while b > 0 and arr[b - 1] > arr[b]:
            arr[b - 1], arr[b] = arr[b], arr[b - 1]
            sgn = -sgn
            b -= 1
    return sgn, arr


def _mul_blades(x, y):
    sgn, arr = _perm_sign(list(x) + list(y))
    out, i = [], 0
    while i < len(arr):
        if i + 1 < len(arr) and arr[i] == arr[i + 1]:
            if arr[i] == 0:
                return 0.0, ()
            i += 2
        else:
            out.append(arr[i])
            i += 1
    return sgn, tuple(out)


def _tables():
    gp = np.zeros((16, 16, 16), np.float32)
    wedge = np.zeros((16, 16, 16), np.float32)
    for i, a in enumerate(_BLADES):
        for j, b in enumerate(_BLADES):
            s, c = _mul_blades(a, b)
            if s:
                gp[i, j, _B2I[c]] = s
            if not (set(a) & set(b)):
                s2, arr = _perm_sign(list(a) + list(b))
                wedge[i, j, _B2I[tuple(arr)]] = s2
    dual = np.zeros((16, 16), np.float32)
    for i, bl in enumerate(_BLADES):
        comp = tuple(sorted(set((0, 1, 2, 3)) - set(bl)))
        s, _ = _perm_sign(list(bl) + list(comp))
        dual[_B2I[comp], i] = s
    join = np.einsum("mn,pqm,pi,qj->ijn", dual, wedge, dual, dual)
    return gp, join.astype(np.float32)


_GP_TBL, _JOIN_TBL = _tables()


def _term_list(tbl):
    out = [[] for _ in range(16)]
    for i, j, n in np.argwhere(tbl != 0.0):
        out[int(n)].append((int(i), int(j), float(tbl[i, j, n])))
    return out


_GP_TERMS = _term_list(_GP_TBL)
_JOIN_TERMS = _term_list(_JOIN_TBL)


def _block_kernel(x_ref, w_ref, b_ref, mask_ref, qs_ref, ks_ref, wtri_ref,
                  m1_ref, wo_ref, bo_ref, wb_ref,
                  bb_ref, ps_ref, w1_ref, b1_ref, wsc_ref, bsc_ref, bc_ref,
                  w2_ref, b2_ref, o_ref, *,
                  heads, cdim, inv_c, seq, scale):
    bf = jnp.bfloat16
    x = x_ref[0]
    ms = jnp.sum(x * x * mask_ref[...], axis=-1, keepdims=True) * inv_c
    xn = (x * jax.lax.rsqrt(ms + RMS_EPS)).astype(bf)
    qkv = jnp.dot(xn, w_ref[...], preferred_element_type=jnp.float32)
    qkv = qkv + b_ref[...]

    half = seq // 2
    row = jax.lax.broadcasted_iota(jnp.int32, (half, half), 0)
    col = jax.lax.broadcasted_iota(jnp.int32, (half, half), 1)
    causal_ll = col <= row

    def masked_softmax(s, causal):
        s = jnp.where(causal, s, -1e30)
        m = jnp.max(s, axis=-1, keepdims=True)
        p = jnp.exp(s - m)
        l = jnp.sum(p, axis=-1, keepdims=True)
        return p.astype(bf), l

    parts = []
    for h in range(heads):
        q = qkv[:, h * cdim:(h + 1) * cdim]
        k = qkv[:, (heads + h) * cdim:(heads + h + 1) * cdim]
        v = qkv[:, (2 * heads + h) * cdim:(2 * heads + h + 1) * cdim]
        cv = jnp.sum((k * k) * wtri_ref[h][None, :], axis=-1, keepdims=True)
        qf = (q * qs_ref[h][None, :] + m1_ref[...]).astype(bf)
        kf = (k * ks_ref[...] + cv * m1_ref[...]).astype(bf)
        vb = v.astype(bf)
        qf_lo, qf_hi = qf[:half], qf[half:]
        kf_lo = kf[:half]
        s_lo = jax.lax.dot_general(qf_lo, kf_lo, (((1,), (1,)), ((), ())),
                                   preferred_element_type=jnp.float32) * scale
        p_lo, l_lo = masked_softmax(s_lo, causal_ll)
        a_lo = jnp.dot(p_lo, vb[:half],
                       preferred_element_type=jnp.float32) * (1.0 / l_lo)
        s_hi = jax.lax.dot_general(qf_hi, kf, (((1,), (1,)), ((), ())),
                                   preferred_element_type=jnp.float32) * scale
        hi_mask = jnp.concatenate(
            [jnp.ones((half, half), jnp.bool_), causal_ll], axis=1)
        p_hi, l_hi = masked_softmax(s_hi, hi_mask)
        a_hi = jnp.dot(p_hi, vb,
                       preferred_element_type=jnp.float32) * (1.0 / l_hi)
        attn_h = jnp.concatenate([a_lo, a_hi], axis=0)
        parts.append(attn_h.astype(bf))

    attn_cat = jnp.concatenate(parts, axis=1)
    acc = jnp.dot(attn_cat, wo_ref[...], preferred_element_type=jnp.float32)
    xa = acc + bo_ref[...] + x
    ms2 = jnp.sum(xa * xa * mask_ref[...], axis=-1, keepdims=True) * inv_c
    xn2 = (xa * jax.lax.rsqrt(ms2 + RMS_EPS)).astype(bf)
    y = jnp.dot(xn2, wb_ref[...], preferred_element_type=jnp.float32)
    y = y + bb_ref[...]

    odim = wb_ref.shape[1] // 4
    c_i = odim // MV

    def to_bm(kk):
        tt = jnp.transpose(y[:, kk * odim:(kk + 1) * odim].astype(bf))
        return tt.reshape(MV, c_i, seq)

    lg, rg, rj = to_bm(0), to_bm(1), to_bm(3)
    lj = to_bm(2) * ps_ref[0, 0, 0].astype(bf)

    zparts = []
    for terms, a, bb2 in ((_GP_TERMS, lg, rg), (_JOIN_TERMS, lj, rj)):
        outs = []
        for n in range(16):
            nacc = None
            for (i, j, sgn) in terms[n]:
                t = a[i] * bb2[j]
                if sgn == -1.0:
                    t = -t
                elif sgn != 1.0:
                    t = t * sgn
                nacc = t if nacc is None else nacc + t
            outs.append(nacc if nacc is not None else jnp.zeros_like(a[0]))
        zp = jnp.stack(outs, axis=0)
        zp = jnp.transpose(zp.reshape(odim, seq))
        zparts.append(zp)
    z = jnp.concatenate(zparts, axis=1)

    z2 = jnp.dot(z, w1_ref[...], preferred_element_type=jnp.float32)
    z2 = z2 + b1_ref[...]
    g32 = jnp.dot(z, wsc_ref[...], preferred_element_type=jnp.float32)
    g32 = jax.nn.gelu(g32 + bsc_ref[...], approximate=True)
    gate = jnp.dot(g32.astype(bf), bc_ref[...],
                   preferred_element_type=jnp.float32)
    gated = (z2 * gate).astype(bf)
    out = jnp.dot(gated, w2_ref[...], preferred_element_type=jnp.float32)
    o_ref[0] = out + b2_ref[...] + xa


def _feature_constants(w_ipa, w_daa, c_h):
    heads = w_ipa.shape[0]
    cdim = c_h * MV
    nd = np.zeros((MV,), np.float32)
    nd[list(ND_LANES)] = 1.0
    tri = np.zeros((MV,), np.float32)
    tri[list(TRI_LANES)] = 1.0

    qscale = (w_ipa[:, :, None] * jnp.asarray(nd)
              + 2.0 * w_daa[:, :, None] * jnp.asarray(tri))
    qscale = qscale.reshape(heads, cdim)
    kscale = np.tile(nd + tri, c_h).reshape(1, cdim)
    wtri = (-w_daa[:, :, None] * jnp.asarray(tri)).reshape(heads, cdim)
    m1 = np.zeros((1, cdim), np.float32)
    m1[0, 1] = 1.0
    return qscale, jnp.asarray(kscale), wtri, jnp.asarray(m1)


def kernel(x, ref_input, qkv_w, qkv_b, attn_out_w, attn_out_b, bil_w, bil_b,
           bil_out_w, bil_out_b, mlp_out_w, mlp_out_b, w_ipa, w_daa,
           norm_mask):
    b, t, c_h, mv = x.shape
    assert mv == MV
    heads = w_ipa.shape[0]
    cdim = c_h * MV
    c_inter = bil_w.shape[1] // (4 * MV)
    inv_c = 1.0 / c_h
    scale = 1.0 / np.sqrt(c_h * 13)

    x3 = x.reshape(b, t, cdim)
    qs, ks, wtri, m1 = _feature_constants(w_ipa, w_daa, c_h)
    ref_ps = jnp.broadcast_to(ref_input[:, 0, 0, 15][:, None, None],
                              (b, 1, 128)).astype(jnp.float32)
    zdim = 2 * c_inter * MV
    wsc = bil_out_w[:, ::MV]
    bsc = bil_out_b[:, ::MV]
    bcast = np.zeros((c_h, cdim), np.float32)
    for c in range(c_h):
        bcast[c, c * MV:(c + 1) * MV] = 1.0
    bcast = jnp.asarray(bcast)
    odim = c_inter * MV
    bil_w_p = bil_w.reshape(cdim, 4, c_inter, MV)
    bil_w_p = jnp.transpose(bil_w_p, (0, 1, 3, 2)).reshape(cdim, 4 * odim)
    bil_b_p = bil_b.reshape(1, 4, c_inter, MV)
    bil_b_p = jnp.transpose(bil_b_p, (0, 1, 3, 2)).reshape(1, 4 * odim)
    w1_p = bil_out_w.reshape(2, c_inter, MV, cdim)
    w1_p = jnp.transpose(w1_p, (0, 2, 1, 3)).reshape(zdim, cdim)
    wsc_p = wsc.reshape(2, c_inter, MV, c_h)
    wsc_p = jnp.transpose(wsc_p, (0, 2, 1, 3)).reshape(zdim, c_h)

    bf = jnp.bfloat16
    out = pl.pallas_call(
        functools.partial(_block_kernel, heads=heads, cdim=cdim, inv_c=inv_c,
                          seq=t, scale=scale),
        out_shape=jax.ShapeDtypeStruct((b, t, cdim), jnp.float32),
        grid=(b,),
        in_specs=[
            pl.BlockSpec((1, t, cdim), lambda i: (i, 0, 0)),
            pl.BlockSpec(qkv_w.shape, lambda i: (0, 0)),
            pl.BlockSpec(qkv_b.shape, lambda i: (0, 0)),
            pl.BlockSpec(norm_mask.shape, lambda i: (0, 0)),
            pl.BlockSpec((heads, cdim), lambda i: (0, 0)),
            pl.BlockSpec((1, cdim), lambda i: (0, 0)),
            pl.BlockSpec((heads, cdim), lambda i: (0, 0)),
            pl.BlockSpec((1, cdim), lambda i: (0, 0)),
            pl.BlockSpec((heads * cdim, cdim), lambda i: (0, 0)),
            pl.BlockSpec(attn_out_b.shape, lambda i: (0, 0)),
            pl.BlockSpec(bil_w.shape, lambda i: (0, 0)),
            pl.BlockSpec(bil_b.shape, lambda i: (0, 0)),
            pl.BlockSpec((1, 1, 128), lambda i: (i, 0, 0)),
            pl.BlockSpec((zdim, cdim), lambda i: (0, 0)),
            pl.BlockSpec((1, cdim), lambda i: (0, 0)),
            pl.BlockSpec((zdim, c_h), lambda i: (0, 0)),
            pl.BlockSpec((1, c_h), lambda i: (0, 0)),
            pl.BlockSpec((c_h, cdim), lambda i: (0, 0)),
            pl.BlockSpec((cdim, cdim), lambda i: (0, 0)),
            pl.BlockSpec(mlp_out_b.shape, lambda i: (0, 0)),
        ],
        out_specs=pl.BlockSpec((1, t, cdim), lambda i: (i, 0, 0)),
        compiler_params=pltpu.CompilerParams(
            dimension_semantics=("parallel",),
            vmem_limit_bytes=100 * 1024 * 1024),
    )(x3, qkv_w.astype(bf), qkv_b, norm_mask, qs, ks, wtri, m1,
      attn_out_w.astype(bf), attn_out_b,
      bil_w_p.astype(bf), bil_b_p, ref_ps, w1_p.astype(bf), bil_out_b,
      wsc_p.astype(bf), bsc, bcast.astype(bf),
      mlp_out_w.astype(bf), mlp_out_b)

    return out.reshape(b, t, c_h, MV)
